# baseline (speedup 1.0000x reference)
import functools

import jax
import jax.numpy as jnp
from jax import lax
from jax.experimental import pallas as pl
from jax.experimental.pallas import tpu as pltpu
from jax.experimental.pallas import tpu_sc as plsc

N = 10000
E = 160000
IN = 256
H = 512
OUT = 256
L = 3
FFN = 1024

MB = 1000
NM = N // MB

NWORK = 32
EW = 5120
E_PAD = NWORK * EW
EBR = 128
NR = 16
RNODE = 10240 // NR
LCAP = 6688
EA = 3632
EB2 = 6608
NP = 10240
NROW = NP // 16

_SC_MESH = plsc.VectorSubcoreMesh(core_axis_name="c", subcore_axis_name="s")


def _layernorm(x, g, b, eps=1e-5):
    m = jnp.mean(x, axis=-1, keepdims=True)
    v = jnp.mean((x - m) ** 2, axis=-1, keepdims=True)
    return (x - m) / jnp.sqrt(v + eps) * g + b


def _zero16():
    return jnp.zeros((16,), jnp.float32)


def _iota16():
    return lax.iota(jnp.int32, 16)


def _sc_wid():
    return lax.axis_index("s") * 2 + lax.axis_index("c")


@functools.partial(
    pl.kernel,
    out_type=jax.ShapeDtypeStruct((NWORK, N), jnp.float32),
    mesh=_SC_MESH,
    compiler_params=pltpu.CompilerParams(needs_layout_passes=False, use_tc_tiling_on_sc=False),
    scratch_types=[
        pltpu.VMEM((N,), jnp.float32),
        pltpu.VMEM((N,), jnp.float32),
        pltpu.VMEM((EW,), jnp.int32),
        pltpu.VMEM((EW,), jnp.int32),
    ],
)
def _sc_neigh(delta_hbm, src_hbm, dst_hbm, out_hbm, delta_v, acc_v, src_v,
              dst_v):
    wid = _sc_wid()
    ebase = wid * EW
    pltpu.sync_copy(delta_hbm, delta_v)
    pltpu.sync_copy(src_hbm.at[pl.ds(ebase, EW)], src_v)
    pltpu.sync_copy(dst_hbm.at[pl.ds(ebase, EW)], dst_v)

    def zero_body(i, _):
        acc_v[pl.ds(i * 16, 16)] = _zero16()
        return 0
    lax.fori_loop(0, N // 16, zero_body, 0)

    iota = _iota16()

    def edge_body(g, _):
        off = g * 16
        sv = src_v[pl.ds(off, 16)]
        dv = dst_v[pl.ds(off, 16)]
        vals = plsc.load_gather(delta_v, [sv])
        mask = (ebase + off + iota) < E
        plsc.addupdate_scatter(acc_v, [dv], vals, mask=mask)
        return 0
    lax.fori_loop(0, EW // 16, edge_body, 0)
    pltpu.sync_copy(acc_v, out_hbm.at[wid])


@functools.partial(
    pl.kernel,
    out_type=[
        jax.ShapeDtypeStruct((E_PAD,), jnp.float32),
        jax.ShapeDtypeStruct((NWORK, N), jnp.float32),
    ],
    mesh=_SC_MESH,
    compiler_params=pltpu.CompilerParams(needs_layout_passes=False, use_tc_tiling_on_sc=False),
    scratch_types=[
        pltpu.VMEM((N,), jnp.float32),
        pltpu.VMEM((N,), jnp.float32),
        pltpu.VMEM((N,), jnp.float32),
        pltpu.VMEM((EW,), jnp.int32),
        pltpu.VMEM((EW,), jnp.int32),
        pltpu.VMEM((EW,), jnp.float32),
    ],
)
def _sc_edge_w(s2_hbm, src_hbm, dst_hbm, w_hbm, out_hbm, sd_v, ss_v, acc_v,
               src_v, dst_v, w_v):
    wid = _sc_wid()
    ebase = wid * EW
    pltpu.sync_copy(s2_hbm.at[0], sd_v)
    pltpu.sync_copy(s2_hbm.at[1], ss_v)
    pltpu.sync_copy(src_hbm.at[pl.ds(ebase, EW)], src_v)
    pltpu.sync_copy(dst_hbm.at[pl.ds(ebase, EW)], dst_v)

    def zero_body(i, _):
        acc_v[pl.ds(i * 16, 16)] = _zero16()
        return 0
    lax.fori_loop(0, N // 16, zero_body, 0)

    iota = _iota16()

    def edge_body(g, _):
        off = g * 16
        sv = src_v[pl.ds(off, 16)]
        dv = dst_v[pl.ds(off, 16)]
        e = (plsc.load_gather(sd_v, [dv]) + plsc.load_gather(ss_v, [sv]))
        e = jnp.where(e >= 0.0, e, 0.2 * e)
        e = jnp.clip(e, -5.0, 5.0)
        w = jnp.exp(e)
        mask = (ebase + off + iota) < E
        w = jnp.where(mask, w, 0.0)
        w_v[pl.ds(off, 16)] = w
        plsc.addupdate_scatter(acc_v, [dv], w)
        return 0
    lax.fori_loop(0, EW // 16, edge_body, 0)
    pltpu.sync_copy(w_v, w_hbm.at[pl.ds(ebase, EW)])
    pltpu.sync_copy(acc_v, out_hbm.at[wid])


@functools.partial(
    pl.kernel,
    out_type=jax.ShapeDtypeStruct((2, NP, H), jnp.float32),
    mesh=_SC_MESH,
    compiler_params=pltpu.CompilerParams(needs_layout_passes=False, use_tc_tiling_on_sc=False),
    scratch_types=[
        pltpu.VMEM_SHARED((RNODE, H), jnp.float32),
        pltpu.VMEM((EB2,), jnp.int32),
        pltpu.VMEM((EB2,), jnp.int32),
        pltpu.VMEM((EB2,), jnp.float32),
        pltpu.VMEM((LCAP,), jnp.int32),
        pltpu.VMEM((LCAP,), jnp.int32),
        pltpu.VMEM((LCAP,), jnp.float32),
        pltpu.VMEM((EBR,), jnp.int32),
        pltpu.VMEM((EBR,), jnp.int32),
        pltpu.VMEM((EBR,), jnp.float32),
        pltpu.VMEM((EBR, H), jnp.float32),
        pltpu.VMEM((8, H), jnp.float32),
        pltpu.SemaphoreType.DMA,
        pltpu.SemaphoreType.DMA,
    ],
)
def _sc_agg(hc_hbm, src_hbm, dst_hbm, w_hbm, out_hbm,
            acc_sh, src_v, dst_v, w_v, lsrc_v, ldst_v, lw_v,
            idxg_v, dstb_v, wb_v, rows_v, zero_v, sem, sem2):
    cid = lax.axis_index("c")
    sid = lax.axis_index("s")
    ebase = jnp.where(cid == 0, sid * EA, 16 * EA + sid * EB2)
    ecnt = jnp.where(cid == 0, EA, EB2)
    nrow = RNODE // 16

    pltpu.sync_copy(src_hbm.at[pl.ds(ebase, EB2)], src_v)
    pltpu.sync_copy(dst_hbm.at[pl.ds(ebase, EB2)], dst_v)
    pltpu.sync_copy(w_hbm.at[pl.ds(ebase, EB2)], w_v)

    def zfill(i, _):
        for c in range(H // 16):
            zero_v[i, pl.ds(c * 16, 16)] = _zero16()
        return 0
    lax.fori_loop(0, 8, zfill, 0)

    iota = _iota16()

    def range_body(r, _):
        lo = r * RNODE

        def compact_body(g, cnt):
            off = g * 16
            sv = src_v[pl.ds(off, 16)]
            dv = dst_v[pl.ds(off, 16)] - lo
            wv = w_v[pl.ds(off, 16)]
            inr = (dv >= 0) & (dv < RNODE)
            plsc.store_compressed(lsrc_v.at[pl.ds(cnt, 16)], sv, mask=inr)
            plsc.store_compressed(ldst_v.at[pl.ds(cnt, 16)], dv, mask=inr)
            plsc.store_compressed(lw_v.at[pl.ds(cnt, 16)], wv, mask=inr)
            return cnt + plsc.all_reduce_population_count(inr)[0]
        cnt = lax.fori_loop(0, ecnt // 16, compact_body, 0)

        for k in range(5):
            pltpu.sync_copy(zero_v, acc_sh.at[pl.ds(sid * nrow + k * 8, 8)])
        plsc.subcore_barrier()

        nb = (cnt + (EBR - 1)) // EBR

        def batch_body(j, _):
            base = j * EBR

            def clean_body(g, _):
                off = base + g * 16
                loc = g * 16 + iota
                ok = (off + iota) < cnt
                idxg_v[pl.ds(g * 16, 16)] = jnp.where(
                    ok, lsrc_v[pl.ds(off, 16)], loc)
                dstb_v[pl.ds(g * 16, 16)] = jnp.where(
                    ok, ldst_v[pl.ds(off, 16)], loc)
                wb_v[pl.ds(g * 16, 16)] = jnp.where(
                    ok, lw_v[pl.ds(off, 16)], 0.0)
                return 0
            lax.fori_loop(0, EBR // 16, clean_body, 0)

            hbr = EBR // 2
            cp1 = pltpu.async_copy(
                hc_hbm.at[idxg_v.at[pl.ds(0, hbr)]],
                rows_v.at[pl.ds(0, hbr)], sem)
            cp2 = pltpu.async_copy(
                hc_hbm.at[idxg_v.at[pl.ds(hbr, hbr)]],
                rows_v.at[pl.ds(hbr, hbr)], sem2)
            def scale_body(g, _):
                wvec = wb_v[pl.ds(g * 16, 16)]
                for lane in range(16):
                    b = g * 16 + lane
                    w = wvec[lane]
                    for c in range(H // 16):
                        rows_v[b, pl.ds(c * 16, 16)] = (
                            rows_v[b, pl.ds(c * 16, 16)] * w)
                return 0
            cp1.wait()
            lax.fori_loop(0, EBR // 32, scale_body, 0)
            cp2.wait()
            lax.fori_loop(EBR // 32, EBR // 16, scale_body, 0)
            pltpu.sync_copy(rows_v, acc_sh.at[dstb_v], add=True)
            return 0
        lax.fori_loop(0, nb, batch_body, 0)

        plsc.subcore_barrier()
        pltpu.sync_copy(acc_sh.at[pl.ds(sid * nrow, nrow)],
                        out_hbm.at[cid, pl.ds(lo + sid * nrow, nrow)])
        plsc.subcore_barrier()
        return 0
    lax.fori_loop(0, NR, range_body, 0)


def _k1_body(x_ref, w_ref, b_ref, w4_ref, h_ref, st_ref):
    h = jnp.maximum(x_ref[...] @ w_ref[...] + b_ref[...], 0.0)
    h_ref[...] = h
    delta = jnp.sum(jnp.abs(h), axis=1, keepdims=True)
    s3 = h @ w4_ref[...]
    st_ref[...] = jnp.concatenate(
        [delta, s3, jnp.zeros((h.shape[0], 4), jnp.float32)], axis=1)


def _input_proj(x, in_w, in_b, w4):
    return pl.pallas_call(
        _k1_body,
        grid=(NM,),
        in_specs=[
            pl.BlockSpec((MB, IN), lambda m: (m, 0)),
            pl.BlockSpec((IN, H), lambda m: (0, 0)),
            pl.BlockSpec((H,), lambda m: (0,)),
            pl.BlockSpec((H, 3), lambda m: (0, 0)),
        ],
        out_specs=[
            pl.BlockSpec((MB, H), lambda m: (m, 0)),
            pl.BlockSpec((MB, 8), lambda m: (m, 0)),
        ],
        out_shape=[
            jax.ShapeDtypeStruct((N, H), jnp.float32),
            jax.ShapeDtypeStruct((N, 8), jnp.float32),
        ],
    )(x, in_w, in_b, w4)


def _k3_body(st_ref, np_ref, wb_ref, s2_ref):
    neigh = jnp.sum(np_ref[...], axis=0)
    st = st_ref[...]
    pi = jax.nn.sigmoid(st[:, 1] + neigh)
    wp_tail = wb_ref[0, 0]
    att_b = wb_ref[0, 1]
    s_dst = st[:, 2]
    s_src = st[:, 3] + pi * wp_tail + att_b
    s2_ref[...] = jnp.stack([s_dst, s_src], axis=0)


def _node_scores(stats, neigh_partials, wb):
    return pl.pallas_call(
        _k3_body,
        grid=(1,),
        in_specs=[
            pl.BlockSpec((N, 8), lambda i: (0, 0)),
            pl.BlockSpec(neigh_partials.shape, lambda i: (0, 0)),
            pl.BlockSpec((1, 2), lambda i: (0, 0)),
        ],
        out_specs=pl.BlockSpec((2, N), lambda i: (0, 0)),
        out_shape=jax.ShapeDtypeStruct((2, N), jnp.float32),
    )(stats, neigh_partials, wb)


def _k5_body(dp_ref, out_ref):
    denom = jnp.sum(dp_ref[...], axis=0)
    out_ref[...] = (1.0 / (denom + 1e-16))[:, None]


def _denom_inv(denom_partials):
    return pl.pallas_call(
        _k5_body,
        grid=(1,),
        in_specs=[pl.BlockSpec(denom_partials.shape, lambda i: (0, 0))],
        out_specs=pl.BlockSpec((N, 1), lambda i: (0, 0)),
        out_shape=jax.ShapeDtypeStruct((N, 1), jnp.float32),
    )(denom_partials)


def _ka_body(h_ref, g_ref, b_ref, w_ref, cb_ref, hc_ref):
    hn = _layernorm(h_ref[...], g_ref[...], b_ref[...])
    hc_ref[...] = hn @ w_ref[...] + cb_ref[...]


def _conv_chunks(h, g, b, w, cb):
    return pl.pallas_call(
        _ka_body,
        grid=(NM,),
        in_specs=[
            pl.BlockSpec((MB, H), lambda m: (m, 0)),
            pl.BlockSpec((H,), lambda m: (0,)),
            pl.BlockSpec((H,), lambda m: (0,)),
            pl.BlockSpec((H, H), lambda m: (0, 0)),
            pl.BlockSpec((H,), lambda m: (0,)),
        ],
        out_specs=pl.BlockSpec((MB, H), lambda m: (m, 0)),
        out_shape=jax.ShapeDtypeStruct((N, H), jnp.float32),
    )(h, g, b, w, cb)


def _kc_body(h_ref, p_ref, di_ref, g1_ref, b1_ref,
             w1_ref, fb1_ref, w2_ref, fb2_ref, hout_ref):
    p = p_ref[...]
    agg = (p[0] + p[1]) * di_ref[...]
    h = h_ref[...] + jnp.maximum(agg, 0.0)
    hn = _layernorm(h, g1_ref[...], b1_ref[...])
    u = hn @ w1_ref[...] + fb1_ref[...]
    u = 0.5 * u * (1.0 + lax.erf(u * 0.7071067811865476))
    h = h + u @ w2_ref[...] + fb2_ref[...]
    hout_ref[...] = h


def _post_agg_ffn(h, partials, denom_inv, blk):
    return pl.pallas_call(
        _kc_body,
        grid=(NM,),
        in_specs=[
            pl.BlockSpec((MB, H), lambda m: (m, 0)),
            pl.BlockSpec((2, MB, H), lambda m: (0, m, 0)),
            pl.BlockSpec((MB, 1), lambda m: (m, 0)),
            pl.BlockSpec((H,), lambda m: (0,)),
            pl.BlockSpec((H,), lambda m: (0,)),
            pl.BlockSpec((H, FFN), lambda m: (0, 0)),
            pl.BlockSpec((FFN,), lambda m: (0,)),
            pl.BlockSpec((FFN, H), lambda m: (0, 0)),
            pl.BlockSpec((H,), lambda m: (0,)),
        ],
        out_specs=pl.BlockSpec((MB, H), lambda m: (m, 0)),
        out_shape=jax.ShapeDtypeStruct((N, H), jnp.float32),
    )(h, partials, denom_inv, blk['ln2_g'], blk['ln2_b'],
      blk['ffn_w1'], blk['ffn_b1'], blk['ffn_w2'], blk['ffn_b2'])


def _k7_body(h_ref, w_ref, b_ref, o_ref):
    o_ref[...] = h_ref[...] @ w_ref[...] + b_ref[...]


def _out_proj(h, w, b):
    return pl.pallas_call(
        _k7_body,
        grid=(NM,),
        in_specs=[
            pl.BlockSpec((MB, H), lambda m: (m, 0)),
            pl.BlockSpec((H, OUT), lambda m: (0, 0)),
            pl.BlockSpec((OUT,), lambda m: (0,)),
        ],
        out_specs=pl.BlockSpec((MB, OUT), lambda m: (m, 0)),
        out_shape=jax.ShapeDtypeStruct((N, OUT), jnp.float32),
    )(h, w, b)


def kernel(x, edge_index, params):
    src = jnp.pad(edge_index[0], (0, E_PAD - E))
    dst = jnp.pad(edge_index[1], (0, E_PAD - E))

    att_w = params['att_w'][:, 0]
    w4 = jnp.stack([params['wp'][:, 0], att_w[:H], att_w[H:2 * H]], axis=1)
    h, stats = _input_proj(x, params['in_w'], params['in_b'], w4)

    delta = stats[:, 0]
    neigh_partials = _sc_neigh(delta, src, dst)
    s2 = _node_scores(stats, neigh_partials,
                      jnp.array([[att_w[2 * H], params['att_b'][0]]],
                                jnp.float32))

    w_e, denom_partials = _sc_edge_w(s2, src, dst)
    denom_inv = _denom_inv(denom_partials)

    for l in range(L):
        blk = params['blocks'][l]
        hc = _conv_chunks(h, blk['ln1_g'], blk['ln1_b'],
                          blk['conv_w'], blk['conv_b'])
        partials = _sc_agg(hc, src, dst, w_e)
        h = _post_agg_ffn(h, partials, denom_inv, blk)

    return _out_proj(h, params['out_w'], params['out_b'])


# speedup vs baseline: 1.1406x; 1.1406x over previous
import functools

import jax
import jax.numpy as jnp
from jax import lax
from jax.experimental import pallas as pl
from jax.experimental.pallas import tpu as pltpu
from jax.experimental.pallas import tpu_sc as plsc

N = 10000
E = 160000
IN = 256
H = 512
OUT = 256
L = 3
FFN = 1024

MB = 1000
NM = N // MB

NWORK = 32
EW = 5120
E_PAD = NWORK * EW
EBR = 128
NR = 16
RNODE = 10240 // NR
LCAP = EW + 16
NP = 10240
NROW = NP // 16

_SC_MESH = plsc.VectorSubcoreMesh(core_axis_name="c", subcore_axis_name="s")


def _layernorm(x, g, b, eps=1e-5):
    m = jnp.mean(x, axis=-1, keepdims=True)
    v = jnp.mean((x - m) ** 2, axis=-1, keepdims=True)
    return (x - m) / jnp.sqrt(v + eps) * g + b


def _zero16():
    return jnp.zeros((16,), jnp.float32)


def _iota16():
    return lax.iota(jnp.int32, 16)


def _sc_wid():
    return lax.axis_index("s") * 2 + lax.axis_index("c")


@functools.partial(
    pl.kernel,
    out_type=jax.ShapeDtypeStruct((NWORK, N), jnp.float32),
    mesh=_SC_MESH,
    compiler_params=pltpu.CompilerParams(needs_layout_passes=False, use_tc_tiling_on_sc=False),
    scratch_types=[
        pltpu.VMEM((N,), jnp.float32),
        pltpu.VMEM((N,), jnp.float32),
        pltpu.VMEM((EW,), jnp.int32),
        pltpu.VMEM((EW,), jnp.int32),
    ],
)
def _sc_neigh(delta_hbm, src_hbm, dst_hbm, out_hbm, delta_v, acc_v, src_v,
              dst_v):
    wid = _sc_wid()
    ebase = wid * EW
    pltpu.sync_copy(delta_hbm, delta_v)
    pltpu.sync_copy(src_hbm.at[pl.ds(ebase, EW)], src_v)
    pltpu.sync_copy(dst_hbm.at[pl.ds(ebase, EW)], dst_v)

    def zero_body(i, _):
        acc_v[pl.ds(i * 16, 16)] = _zero16()
        return 0
    lax.fori_loop(0, N // 16, zero_body, 0)

    iota = _iota16()

    def edge_body(g, _):
        off = g * 16
        sv = src_v[pl.ds(off, 16)]
        dv = dst_v[pl.ds(off, 16)]
        vals = plsc.load_gather(delta_v, [sv])
        mask = (ebase + off + iota) < E
        plsc.addupdate_scatter(acc_v, [dv], vals, mask=mask)
        return 0
    lax.fori_loop(0, EW // 16, edge_body, 0)
    pltpu.sync_copy(acc_v, out_hbm.at[wid])


@functools.partial(
    pl.kernel,
    out_type=[
        jax.ShapeDtypeStruct((E_PAD,), jnp.float32),
        jax.ShapeDtypeStruct((NWORK, N), jnp.float32),
    ],
    mesh=_SC_MESH,
    compiler_params=pltpu.CompilerParams(needs_layout_passes=False, use_tc_tiling_on_sc=False),
    scratch_types=[
        pltpu.VMEM((N,), jnp.float32),
        pltpu.VMEM((N,), jnp.float32),
        pltpu.VMEM((N,), jnp.float32),
        pltpu.VMEM((EW,), jnp.int32),
        pltpu.VMEM((EW,), jnp.int32),
        pltpu.VMEM((EW,), jnp.float32),
    ],
)
def _sc_edge_w(s2_hbm, src_hbm, dst_hbm, w_hbm, out_hbm, sd_v, ss_v, acc_v,
               src_v, dst_v, w_v):
    wid = _sc_wid()
    ebase = wid * EW
    pltpu.sync_copy(s2_hbm.at[0], sd_v)
    pltpu.sync_copy(s2_hbm.at[1], ss_v)
    pltpu.sync_copy(src_hbm.at[pl.ds(ebase, EW)], src_v)
    pltpu.sync_copy(dst_hbm.at[pl.ds(ebase, EW)], dst_v)

    def zero_body(i, _):
        acc_v[pl.ds(i * 16, 16)] = _zero16()
        return 0
    lax.fori_loop(0, N // 16, zero_body, 0)

    iota = _iota16()

    def edge_body(g, _):
        off = g * 16
        sv = src_v[pl.ds(off, 16)]
        dv = dst_v[pl.ds(off, 16)]
        e = (plsc.load_gather(sd_v, [dv]) + plsc.load_gather(ss_v, [sv]))
        e = jnp.where(e >= 0.0, e, 0.2 * e)
        e = jnp.clip(e, -5.0, 5.0)
        w = jnp.exp(e)
        mask = (ebase + off + iota) < E
        w = jnp.where(mask, w, 0.0)
        w_v[pl.ds(off, 16)] = w
        plsc.addupdate_scatter(acc_v, [dv], w)
        return 0
    lax.fori_loop(0, EW // 16, edge_body, 0)
    pltpu.sync_copy(w_v, w_hbm.at[pl.ds(ebase, EW)])
    pltpu.sync_copy(acc_v, out_hbm.at[wid])


@functools.partial(
    pl.kernel,
    out_type=jax.ShapeDtypeStruct((2, NP, H), jnp.float32),
    mesh=_SC_MESH,
    compiler_params=pltpu.CompilerParams(needs_layout_passes=False, use_tc_tiling_on_sc=False),
    scratch_types=[
        pltpu.VMEM_SHARED((RNODE, H), jnp.float32),
        pltpu.VMEM((EW,), jnp.int32),
        pltpu.VMEM((EW,), jnp.int32),
        pltpu.VMEM((EW,), jnp.float32),
        pltpu.VMEM((LCAP,), jnp.int32),
        pltpu.VMEM((LCAP,), jnp.int32),
        pltpu.VMEM((LCAP,), jnp.float32),
        pltpu.VMEM((EBR,), jnp.int32),
        pltpu.VMEM((EBR,), jnp.int32),
        pltpu.VMEM((EBR,), jnp.float32),
        pltpu.VMEM((EBR, H), jnp.float32),
        pltpu.VMEM((8, H), jnp.float32),
        pltpu.SemaphoreType.DMA,
        pltpu.SemaphoreType.DMA,
    ],
)
def _sc_agg(hc_hbm, src_hbm, dst_hbm, w_hbm, out_hbm,
            acc_sh, src_v, dst_v, w_v, lsrc_v, ldst_v, lw_v,
            idxg_v, dstb_v, wb_v, rows_v, zero_v, sem, sem2):
    cid = lax.axis_index("c")
    sid = lax.axis_index("s")
    wid = sid * 2 + cid
    ebase = wid * EW
    nrow = RNODE // 16

    pltpu.sync_copy(src_hbm.at[pl.ds(ebase, EW)], src_v)
    pltpu.sync_copy(dst_hbm.at[pl.ds(ebase, EW)], dst_v)
    pltpu.sync_copy(w_hbm.at[pl.ds(ebase, EW)], w_v)

    def zfill(i, _):
        for c in range(H // 16):
            zero_v[i, pl.ds(c * 16, 16)] = _zero16()
        return 0
    lax.fori_loop(0, 8, zfill, 0)

    iota = _iota16()

    def range_body(r, _):
        lo = r * RNODE

        def compact_body(g, cnt):
            off = g * 16
            sv = src_v[pl.ds(off, 16)]
            dv = dst_v[pl.ds(off, 16)] - lo
            wv = w_v[pl.ds(off, 16)]
            inr = (dv >= 0) & (dv < RNODE)
            plsc.store_compressed(lsrc_v.at[pl.ds(cnt, 16)], sv, mask=inr)
            plsc.store_compressed(ldst_v.at[pl.ds(cnt, 16)], dv, mask=inr)
            plsc.store_compressed(lw_v.at[pl.ds(cnt, 16)], wv, mask=inr)
            return cnt + plsc.all_reduce_population_count(inr)[0]
        cnt = lax.fori_loop(0, EW // 16, compact_body, 0)

        for k in range(5):
            pltpu.sync_copy(zero_v, acc_sh.at[pl.ds(sid * nrow + k * 8, 8)])
        plsc.subcore_barrier()

        nb = (cnt + (EBR - 1)) // EBR

        def batch_body(j, _):
            base = j * EBR

            def clean_body(g, _):
                off = base + g * 16
                loc = g * 16 + iota
                ok = (off + iota) < cnt
                idxg_v[pl.ds(g * 16, 16)] = jnp.where(
                    ok, lsrc_v[pl.ds(off, 16)], loc)
                dstb_v[pl.ds(g * 16, 16)] = jnp.where(
                    ok, ldst_v[pl.ds(off, 16)], loc)
                wb_v[pl.ds(g * 16, 16)] = jnp.where(
                    ok, lw_v[pl.ds(off, 16)], 0.0)
                return 0
            lax.fori_loop(0, EBR // 16, clean_body, 0)

            hbr = EBR // 2
            cp1 = pltpu.async_copy(
                hc_hbm.at[idxg_v.at[pl.ds(0, hbr)]],
                rows_v.at[pl.ds(0, hbr)], sem)
            cp2 = pltpu.async_copy(
                hc_hbm.at[idxg_v.at[pl.ds(hbr, hbr)]],
                rows_v.at[pl.ds(hbr, hbr)], sem2)
            def scale_body(g, _):
                wvec = wb_v[pl.ds(g * 16, 16)]
                for lane in range(16):
                    b = g * 16 + lane
                    w = wvec[lane]
                    for c in range(H // 16):
                        rows_v[b, pl.ds(c * 16, 16)] = (
                            rows_v[b, pl.ds(c * 16, 16)] * w)
                return 0
            cp1.wait()
            lax.fori_loop(0, EBR // 32, scale_body, 0)
            cp2.wait()
            lax.fori_loop(EBR // 32, EBR // 16, scale_body, 0)
            pltpu.sync_copy(rows_v, acc_sh.at[dstb_v], add=True)
            return 0
        lax.fori_loop(0, nb, batch_body, 0)

        plsc.subcore_barrier()
        pltpu.sync_copy(acc_sh.at[pl.ds(sid * nrow, nrow)],
                        out_hbm.at[cid, pl.ds(lo + sid * nrow, nrow)])
        plsc.subcore_barrier()
        return 0
    lax.fori_loop(0, NR, range_body, 0)


def _k1_body(x_ref, w_ref, b_ref, w4_ref, h_ref, st_ref):
    h = jnp.maximum(x_ref[...] @ w_ref[...] + b_ref[...], 0.0)
    h_ref[...] = h
    delta = jnp.sum(jnp.abs(h), axis=1, keepdims=True)
    s3 = h @ w4_ref[...]
    st_ref[...] = jnp.concatenate(
        [delta, s3, jnp.zeros((h.shape[0], 4), jnp.float32)], axis=1)


def _input_proj(x, in_w, in_b, w4):
    return pl.pallas_call(
        _k1_body,
        grid=(NM,),
        in_specs=[
            pl.BlockSpec((MB, IN), lambda m: (m, 0)),
            pl.BlockSpec((IN, H), lambda m: (0, 0)),
            pl.BlockSpec((H,), lambda m: (0,)),
            pl.BlockSpec((H, 3), lambda m: (0, 0)),
        ],
        out_specs=[
            pl.BlockSpec((MB, H), lambda m: (m, 0)),
            pl.BlockSpec((MB, 8), lambda m: (m, 0)),
        ],
        out_shape=[
            jax.ShapeDtypeStruct((N, H), jnp.float32),
            jax.ShapeDtypeStruct((N, 8), jnp.float32),
        ],
    )(x, in_w, in_b, w4)


def _k3_body(st_ref, np_ref, wb_ref, s2_ref):
    neigh = jnp.sum(np_ref[...], axis=0)
    st = st_ref[...]
    pi = jax.nn.sigmoid(st[:, 1] + neigh)
    wp_tail = wb_ref[0, 0]
    att_b = wb_ref[0, 1]
    s_dst = st[:, 2]
    s_src = st[:, 3] + pi * wp_tail + att_b
    s2_ref[...] = jnp.stack([s_dst, s_src], axis=0)


def _node_scores(stats, neigh_partials, wb):
    return pl.pallas_call(
        _k3_body,
        grid=(1,),
        in_specs=[
            pl.BlockSpec((N, 8), lambda i: (0, 0)),
            pl.BlockSpec(neigh_partials.shape, lambda i: (0, 0)),
            pl.BlockSpec((1, 2), lambda i: (0, 0)),
        ],
        out_specs=pl.BlockSpec((2, N), lambda i: (0, 0)),
        out_shape=jax.ShapeDtypeStruct((2, N), jnp.float32),
    )(stats, neigh_partials, wb)


def _k5_body(dp_ref, out_ref):
    denom = jnp.sum(dp_ref[...], axis=0)
    out_ref[...] = (1.0 / (denom + 1e-16))[:, None]


def _denom_inv(denom_partials):
    return pl.pallas_call(
        _k5_body,
        grid=(1,),
        in_specs=[pl.BlockSpec(denom_partials.shape, lambda i: (0, 0))],
        out_specs=pl.BlockSpec((N, 1), lambda i: (0, 0)),
        out_shape=jax.ShapeDtypeStruct((N, 1), jnp.float32),
    )(denom_partials)


def _ka_body(h_ref, g_ref, b_ref, w_ref, cb_ref, hc_ref):
    hn = _layernorm(h_ref[...], g_ref[...], b_ref[...])
    hc_ref[...] = hn @ w_ref[...] + cb_ref[...]


def _conv_chunks(h, g, b, w, cb):
    return pl.pallas_call(
        _ka_body,
        grid=(NM,),
        in_specs=[
            pl.BlockSpec((MB, H), lambda m: (m, 0)),
            pl.BlockSpec((H,), lambda m: (0,)),
            pl.BlockSpec((H,), lambda m: (0,)),
            pl.BlockSpec((H, H), lambda m: (0, 0)),
            pl.BlockSpec((H,), lambda m: (0,)),
        ],
        out_specs=pl.BlockSpec((MB, H), lambda m: (m, 0)),
        out_shape=jax.ShapeDtypeStruct((N, H), jnp.float32),
    )(h, g, b, w, cb)


def _kc_body(h_ref, p_ref, di_ref, g1_ref, b1_ref,
             w1_ref, fb1_ref, w2_ref, fb2_ref, hout_ref):
    p = p_ref[...]
    agg = (p[0] + p[1]) * di_ref[...]
    h = h_ref[...] + jnp.maximum(agg, 0.0)
    hn = _layernorm(h, g1_ref[...], b1_ref[...])
    u = hn @ w1_ref[...] + fb1_ref[...]
    u = 0.5 * u * (1.0 + lax.erf(u * 0.7071067811865476))
    h = h + u @ w2_ref[...] + fb2_ref[...]
    hout_ref[...] = h


def _post_agg_ffn(h, partials, denom_inv, blk):
    return pl.pallas_call(
        _kc_body,
        grid=(NM,),
        in_specs=[
            pl.BlockSpec((MB, H), lambda m: (m, 0)),
            pl.BlockSpec((2, MB, H), lambda m: (0, m, 0)),
            pl.BlockSpec((MB, 1), lambda m: (m, 0)),
            pl.BlockSpec((H,), lambda m: (0,)),
            pl.BlockSpec((H,), lambda m: (0,)),
            pl.BlockSpec((H, FFN), lambda m: (0, 0)),
            pl.BlockSpec((FFN,), lambda m: (0,)),
            pl.BlockSpec((FFN, H), lambda m: (0, 0)),
            pl.BlockSpec((H,), lambda m: (0,)),
        ],
        out_specs=pl.BlockSpec((MB, H), lambda m: (m, 0)),
        out_shape=jax.ShapeDtypeStruct((N, H), jnp.float32),
    )(h, partials, denom_inv, blk['ln2_g'], blk['ln2_b'],
      blk['ffn_w1'], blk['ffn_b1'], blk['ffn_w2'], blk['ffn_b2'])


def _k7_body(h_ref, w_ref, b_ref, o_ref):
    o_ref[...] = h_ref[...] @ w_ref[...] + b_ref[...]


def _out_proj(h, w, b):
    return pl.pallas_call(
        _k7_body,
        grid=(NM,),
        in_specs=[
            pl.BlockSpec((MB, H), lambda m: (m, 0)),
            pl.BlockSpec((H, OUT), lambda m: (0, 0)),
            pl.BlockSpec((OUT,), lambda m: (0,)),
        ],
        out_specs=pl.BlockSpec((MB, OUT), lambda m: (m, 0)),
        out_shape=jax.ShapeDtypeStruct((N, OUT), jnp.float32),
    )(h, w, b)


def kernel(x, edge_index, params):
    src = jnp.pad(edge_index[0], (0, E_PAD - E))
    dst = jnp.pad(edge_index[1], (0, E_PAD - E))

    att_w = params['att_w'][:, 0]
    w4 = jnp.stack([params['wp'][:, 0], att_w[:H], att_w[H:2 * H]], axis=1)
    h, stats = _input_proj(x, params['in_w'], params['in_b'], w4)

    delta = stats[:, 0]
    neigh_partials = _sc_neigh(delta, src, dst)
    s2 = _node_scores(stats, neigh_partials,
                      jnp.array([[att_w[2 * H], params['att_b'][0]]],
                                jnp.float32))

    w_e, denom_partials = _sc_edge_w(s2, src, dst)
    denom_inv = _denom_inv(denom_partials)

    for l in range(L):
        blk = params['blocks'][l]
        hc = _conv_chunks(h, blk['ln1_g'], blk['ln1_b'],
                          blk['conv_w'], blk['conv_b'])
        partials = _sc_agg(hc, src, dst, w_e)
        h = _post_agg_ffn(h, partials, denom_inv, blk)

    return _out_proj(h, params['out_w'], params['out_b'])


# speedup vs baseline: 1.1503x; 1.0085x over previous
import functools

import jax
import jax.numpy as jnp
from jax import lax
from jax.experimental import pallas as pl
from jax.experimental.pallas import tpu as pltpu
from jax.experimental.pallas import tpu_sc as plsc

N = 10000
E = 160000
IN = 256
H = 512
OUT = 256
L = 3
FFN = 1024

MB = 1000
NM = N // MB

NWORK = 32
EW = 5120
E_PAD = NWORK * EW
EBR = 128
NR = 16
RNODE = 10240 // NR
LCAP = EW + 16
NP = 10240
NROW = NP // 16

_SC_MESH = plsc.VectorSubcoreMesh(core_axis_name="c", subcore_axis_name="s")


def _layernorm(x, g, b, eps=1e-5):
    m = jnp.mean(x, axis=-1, keepdims=True)
    v = jnp.mean((x - m) ** 2, axis=-1, keepdims=True)
    return (x - m) / jnp.sqrt(v + eps) * g + b


def _zero16():
    return jnp.zeros((16,), jnp.float32)


def _iota16():
    return lax.iota(jnp.int32, 16)


def _sc_wid():
    return lax.axis_index("s") * 2 + lax.axis_index("c")


@functools.partial(
    pl.kernel,
    out_type=jax.ShapeDtypeStruct((NWORK, N), jnp.float32),
    mesh=_SC_MESH,
    compiler_params=pltpu.CompilerParams(needs_layout_passes=False, use_tc_tiling_on_sc=False),
    scratch_types=[
        pltpu.VMEM((N,), jnp.float32),
        pltpu.VMEM((N,), jnp.float32),
        pltpu.VMEM((EW,), jnp.int32),
        pltpu.VMEM((EW,), jnp.int32),
    ],
)
def _sc_neigh(delta_hbm, src_hbm, dst_hbm, out_hbm, delta_v, acc_v, src_v,
              dst_v):
    wid = _sc_wid()
    ebase = wid * EW
    pltpu.sync_copy(delta_hbm, delta_v)
    pltpu.sync_copy(src_hbm.at[pl.ds(ebase, EW)], src_v)
    pltpu.sync_copy(dst_hbm.at[pl.ds(ebase, EW)], dst_v)

    def zero_body(i, _):
        acc_v[pl.ds(i * 16, 16)] = _zero16()
        return 0
    lax.fori_loop(0, N // 16, zero_body, 0)

    iota = _iota16()

    def edge_body(g, _):
        off = g * 16
        sv = src_v[pl.ds(off, 16)]
        dv = dst_v[pl.ds(off, 16)]
        vals = plsc.load_gather(delta_v, [sv])
        mask = (ebase + off + iota) < E
        plsc.addupdate_scatter(acc_v, [dv], vals, mask=mask)
        return 0
    lax.fori_loop(0, EW // 16, edge_body, 0)
    pltpu.sync_copy(acc_v, out_hbm.at[wid])


@functools.partial(
    pl.kernel,
    out_type=[
        jax.ShapeDtypeStruct((E_PAD,), jnp.float32),
        jax.ShapeDtypeStruct((NWORK, N), jnp.float32),
    ],
    mesh=_SC_MESH,
    compiler_params=pltpu.CompilerParams(needs_layout_passes=False, use_tc_tiling_on_sc=False),
    scratch_types=[
        pltpu.VMEM((N,), jnp.float32),
        pltpu.VMEM((N,), jnp.float32),
        pltpu.VMEM((N,), jnp.float32),
        pltpu.VMEM((EW,), jnp.int32),
        pltpu.VMEM((EW,), jnp.int32),
        pltpu.VMEM((EW,), jnp.float32),
    ],
)
def _sc_edge_w(s2_hbm, src_hbm, dst_hbm, w_hbm, out_hbm, sd_v, ss_v, acc_v,
               src_v, dst_v, w_v):
    wid = _sc_wid()
    ebase = wid * EW
    pltpu.sync_copy(s2_hbm.at[0], sd_v)
    pltpu.sync_copy(s2_hbm.at[1], ss_v)
    pltpu.sync_copy(src_hbm.at[pl.ds(ebase, EW)], src_v)
    pltpu.sync_copy(dst_hbm.at[pl.ds(ebase, EW)], dst_v)

    def zero_body(i, _):
        acc_v[pl.ds(i * 16, 16)] = _zero16()
        return 0
    lax.fori_loop(0, N // 16, zero_body, 0)

    iota = _iota16()

    def edge_body(g, _):
        off = g * 16
        sv = src_v[pl.ds(off, 16)]
        dv = dst_v[pl.ds(off, 16)]
        e = (plsc.load_gather(sd_v, [dv]) + plsc.load_gather(ss_v, [sv]))
        e = jnp.where(e >= 0.0, e, 0.2 * e)
        e = jnp.clip(e, -5.0, 5.0)
        w = jnp.exp(e)
        mask = (ebase + off + iota) < E
        w = jnp.where(mask, w, 0.0)
        w_v[pl.ds(off, 16)] = w
        plsc.addupdate_scatter(acc_v, [dv], w)
        return 0
    lax.fori_loop(0, EW // 16, edge_body, 0)
    pltpu.sync_copy(w_v, w_hbm.at[pl.ds(ebase, EW)])
    pltpu.sync_copy(acc_v, out_hbm.at[wid])


@functools.partial(
    pl.kernel,
    out_type=jax.ShapeDtypeStruct((2, NP, H), jnp.float32),
    mesh=_SC_MESH,
    compiler_params=pltpu.CompilerParams(needs_layout_passes=False, use_tc_tiling_on_sc=False),
    scratch_types=[
        pltpu.VMEM_SHARED((RNODE, H), jnp.float32),
        pltpu.VMEM((EW,), jnp.int32),
        pltpu.VMEM((EW,), jnp.int32),
        pltpu.VMEM((EW,), jnp.float32),
        pltpu.VMEM((LCAP,), jnp.int32),
        pltpu.VMEM((LCAP,), jnp.int32),
        pltpu.VMEM((LCAP,), jnp.float32),
        pltpu.VMEM((EBR,), jnp.int32),
        pltpu.VMEM((EBR,), jnp.int32),
        pltpu.VMEM((EBR,), jnp.float32),
        pltpu.VMEM((EBR, H), jnp.float32),
        pltpu.VMEM((8, H), jnp.float32),
        pltpu.SemaphoreType.DMA,
        pltpu.SemaphoreType.DMA,
    ],
)
def _sc_agg(hc_hbm, src_hbm, dst_hbm, w_hbm, out_hbm,
            acc_sh, src_v, dst_v, w_v, lsrc_v, ldst_v, lw_v,
            idxg_v, dstb_v, wb_v, rows_v, zero_v, sem, sem2):
    cid = lax.axis_index("c")
    sid = lax.axis_index("s")
    wid = sid * 2 + cid
    ebase = wid * EW
    nrow = RNODE // 16

    pltpu.sync_copy(src_hbm.at[pl.ds(ebase, EW)], src_v)
    pltpu.sync_copy(dst_hbm.at[pl.ds(ebase, EW)], dst_v)
    pltpu.sync_copy(w_hbm.at[pl.ds(ebase, EW)], w_v)

    def zfill(i, _):
        for c in range(H // 16):
            zero_v[i, pl.ds(c * 16, 16)] = _zero16()
        return 0
    lax.fori_loop(0, 8, zfill, 0)

    iota = _iota16()

    def range_body(r, _):
        lo = r * RNODE

        def compact_body(g, cnt):
            off = g * 16
            sv = src_v[pl.ds(off, 16)]
            dv = dst_v[pl.ds(off, 16)] - lo
            wv = w_v[pl.ds(off, 16)]
            inr = (dv >= 0) & (dv < RNODE)
            plsc.store_compressed(lsrc_v.at[pl.ds(cnt, 16)], sv, mask=inr)
            plsc.store_compressed(ldst_v.at[pl.ds(cnt, 16)], dv, mask=inr)
            plsc.store_compressed(lw_v.at[pl.ds(cnt, 16)], wv, mask=inr)
            return cnt + plsc.all_reduce_population_count(inr)[0]
        cnt = lax.fori_loop(0, EW // 16, compact_body, 0)

        for k in range(5):
            pltpu.sync_copy(zero_v, acc_sh.at[pl.ds(sid * nrow + k * 8, 8)])
        plsc.subcore_barrier()

        nb = (cnt + (EBR - 1)) // EBR

        def batch_body(j, _):
            base = j * EBR

            def clean_body(g, _):
                off = base + g * 16
                loc = g * 16 + iota
                ok = (off + iota) < cnt
                idxg_v[pl.ds(g * 16, 16)] = jnp.where(
                    ok, lsrc_v[pl.ds(off, 16)], loc)
                dstb_v[pl.ds(g * 16, 16)] = jnp.where(
                    ok, ldst_v[pl.ds(off, 16)], loc)
                wb_v[pl.ds(g * 16, 16)] = jnp.where(
                    ok, lw_v[pl.ds(off, 16)], 0.0)
                return 0
            lax.fori_loop(0, EBR // 16, clean_body, 0)

            hbr = EBR // 2
            cp1 = pltpu.async_copy(
                hc_hbm.at[idxg_v.at[pl.ds(0, hbr)]],
                rows_v.at[pl.ds(0, hbr)], sem)
            cp2 = pltpu.async_copy(
                hc_hbm.at[idxg_v.at[pl.ds(hbr, hbr)]],
                rows_v.at[pl.ds(hbr, hbr)], sem2)
            def scale_body(g, _):
                wvec = wb_v[pl.ds(g * 16, 16)]
                for lane in range(16):
                    b = g * 16 + lane
                    w = wvec[lane]
                    for c in range(H // 16):
                        rows_v[b, pl.ds(c * 16, 16)] = (
                            rows_v[b, pl.ds(c * 16, 16)] * w)
                return 0
            cp1.wait()
            lax.fori_loop(0, EBR // 32, scale_body, 0)
            cp2.wait()
            lax.fori_loop(EBR // 32, EBR // 16, scale_body, 0)
            pltpu.sync_copy(rows_v, acc_sh.at[dstb_v], add=True)
            return 0
        lax.fori_loop(0, nb, batch_body, 0)

        plsc.subcore_barrier()
        pltpu.sync_copy(acc_sh.at[pl.ds(sid * nrow, nrow)],
                        out_hbm.at[cid, pl.ds(lo + sid * nrow, nrow)])
        plsc.subcore_barrier()
        return 0
    lax.fori_loop(0, NR, range_body, 0)


def _k1_body(x_ref, w_ref, b_ref, w4_ref, h_ref, st_ref):
    h = jnp.maximum(x_ref[...] @ w_ref[...] + b_ref[...], 0.0)
    h_ref[...] = h
    delta = jnp.sum(jnp.abs(h), axis=1, keepdims=True)
    s3 = h @ w4_ref[...]
    st_ref[...] = jnp.concatenate(
        [delta, s3, jnp.zeros((h.shape[0], 4), jnp.float32)], axis=1)


def _input_proj(x, in_w, in_b, w4):
    return pl.pallas_call(
        _k1_body,
        grid=(NM,),
        in_specs=[
            pl.BlockSpec((MB, IN), lambda m: (m, 0)),
            pl.BlockSpec((IN, H), lambda m: (0, 0)),
            pl.BlockSpec((H,), lambda m: (0,)),
            pl.BlockSpec((H, 3), lambda m: (0, 0)),
        ],
        out_specs=[
            pl.BlockSpec((MB, H), lambda m: (m, 0)),
            pl.BlockSpec((MB, 8), lambda m: (m, 0)),
        ],
        out_shape=[
            jax.ShapeDtypeStruct((N, H), jnp.float32),
            jax.ShapeDtypeStruct((N, 8), jnp.float32),
        ],
    )(x, in_w, in_b, w4)


def _k3_body(st_ref, np_ref, wb_ref, s2_ref):
    neigh = jnp.sum(np_ref[...], axis=0)
    st = st_ref[...]
    pi = jax.nn.sigmoid(st[:, 1] + neigh)
    wp_tail = wb_ref[0, 0]
    att_b = wb_ref[0, 1]
    s_dst = st[:, 2]
    s_src = st[:, 3] + pi * wp_tail + att_b
    s2_ref[...] = jnp.stack([s_dst, s_src], axis=0)


def _node_scores(stats, neigh_partials, wb):
    return pl.pallas_call(
        _k3_body,
        grid=(1,),
        in_specs=[
            pl.BlockSpec((N, 8), lambda i: (0, 0)),
            pl.BlockSpec(neigh_partials.shape, lambda i: (0, 0)),
            pl.BlockSpec((1, 2), lambda i: (0, 0)),
        ],
        out_specs=pl.BlockSpec((2, N), lambda i: (0, 0)),
        out_shape=jax.ShapeDtypeStruct((2, N), jnp.float32),
    )(stats, neigh_partials, wb)


def _k5_body(dp_ref, out_ref):
    denom = jnp.sum(dp_ref[...], axis=0)
    out_ref[...] = (1.0 / (denom + 1e-16))[:, None]


def _denom_inv(denom_partials):
    return pl.pallas_call(
        _k5_body,
        grid=(1,),
        in_specs=[pl.BlockSpec(denom_partials.shape, lambda i: (0, 0))],
        out_specs=pl.BlockSpec((N, 1), lambda i: (0, 0)),
        out_shape=jax.ShapeDtypeStruct((N, 1), jnp.float32),
    )(denom_partials)


def _ka_body(h_ref, g_ref, b_ref, w_ref, cb_ref, hc_ref):
    hn = _layernorm(h_ref[...], g_ref[...], b_ref[...])
    hc_ref[...] = hn @ w_ref[...] + cb_ref[...]


def _conv_chunks(h, g, b, w, cb):
    return pl.pallas_call(
        _ka_body,
        grid=(NM,),
        in_specs=[
            pl.BlockSpec((MB, H), lambda m: (m, 0)),
            pl.BlockSpec((H,), lambda m: (0,)),
            pl.BlockSpec((H,), lambda m: (0,)),
            pl.BlockSpec((H, H), lambda m: (0, 0)),
            pl.BlockSpec((H,), lambda m: (0,)),
        ],
        out_specs=pl.BlockSpec((MB, H), lambda m: (m, 0)),
        out_shape=jax.ShapeDtypeStruct((N, H), jnp.float32),
    )(h, g, b, w, cb)


def _kc_body(h_ref, p_ref, di_ref, g1_ref, b1_ref,
             w1_ref, fb1_ref, w2_ref, fb2_ref, ow_ref, ob_ref,
             hout_ref, o_ref):
    p = p_ref[...]
    agg = (p[0] + p[1]) * di_ref[...]
    h = h_ref[...] + jnp.maximum(agg, 0.0)
    hn = _layernorm(h, g1_ref[...], b1_ref[...])
    u = hn @ w1_ref[...] + fb1_ref[...]
    u = 0.5 * u * (1.0 + lax.erf(u * 0.7071067811865476))
    h = h + u @ w2_ref[...] + fb2_ref[...]
    hout_ref[...] = h
    o_ref[...] = h @ ow_ref[...] + ob_ref[...]


def _post_agg_ffn(h, partials, denom_inv, blk, ow, ob):
    return pl.pallas_call(
        _kc_body,
        grid=(NM,),
        in_specs=[
            pl.BlockSpec((MB, H), lambda m: (m, 0)),
            pl.BlockSpec((2, MB, H), lambda m: (0, m, 0)),
            pl.BlockSpec((MB, 1), lambda m: (m, 0)),
            pl.BlockSpec((H,), lambda m: (0,)),
            pl.BlockSpec((H,), lambda m: (0,)),
            pl.BlockSpec((H, FFN), lambda m: (0, 0)),
            pl.BlockSpec((FFN,), lambda m: (0,)),
            pl.BlockSpec((FFN, H), lambda m: (0, 0)),
            pl.BlockSpec((H,), lambda m: (0,)),
            pl.BlockSpec((H, OUT), lambda m: (0, 0)),
            pl.BlockSpec((OUT,), lambda m: (0,)),
        ],
        out_specs=[
            pl.BlockSpec((MB, H), lambda m: (m, 0)),
            pl.BlockSpec((MB, OUT), lambda m: (m, 0)),
        ],
        out_shape=[
            jax.ShapeDtypeStruct((N, H), jnp.float32),
            jax.ShapeDtypeStruct((N, OUT), jnp.float32),
        ],
    )(h, partials, denom_inv, blk['ln2_g'], blk['ln2_b'],
      blk['ffn_w1'], blk['ffn_b1'], blk['ffn_w2'], blk['ffn_b2'], ow, ob)


def kernel(x, edge_index, params):
    src = jnp.pad(edge_index[0], (0, E_PAD - E))
    dst = jnp.pad(edge_index[1], (0, E_PAD - E))

    att_w = params['att_w'][:, 0]
    w4 = jnp.stack([params['wp'][:, 0], att_w[:H], att_w[H:2 * H]], axis=1)
    h, stats = _input_proj(x, params['in_w'], params['in_b'], w4)

    delta = stats[:, 0]
    neigh_partials = _sc_neigh(delta, src, dst)
    s2 = _node_scores(stats, neigh_partials,
                      jnp.array([[att_w[2 * H], params['att_b'][0]]],
                                jnp.float32))

    w_e, denom_partials = _sc_edge_w(s2, src, dst)
    denom_inv = _denom_inv(denom_partials)

    for l in range(L):
        blk = params['blocks'][l]
        hc = _conv_chunks(h, blk['ln1_g'], blk['ln1_b'],
                          blk['conv_w'], blk['conv_b'])
        partials = _sc_agg(hc, src, dst, w_e)
        h, out = _post_agg_ffn(h, partials, denom_inv, blk,
                               params['out_w'], params['out_b'])

    return out
